# baseline (speedup 1.0000x reference)
import functools

import jax
import jax.numpy as jnp
from jax import lax
from jax.experimental import pallas as pl
from jax.experimental.pallas import tpu as pltpu
from jax.experimental.pallas import tpu_sc as plsc

NUM_RELATIONS = 1000000
EMBEDDING_DIM = 32
BATCH = 4096
FIELDS = 26

_info = plsc.get_sparse_core_info()
_NC, _NS = _info.num_cores, _info.num_subcores
_NW = _NC * _NS
_NROWS = FIELDS * EMBEDDING_DIM
_RPW = _NROWS // _NW
_NS_W = 8
_CS = 256
_WPR = BATCH // (_NS_W * _CS)
_NWAVE = _RPW * _WPR
_NG = _CS // 16
_BPR = NUM_RELATIONS // 32


@functools.partial(
    pl.kernel,
    out_type=jax.ShapeDtypeStruct((FIELDS, EMBEDDING_DIM, BATCH), jnp.float32),
    mesh=plsc.VectorSubcoreMesh(core_axis_name="c", subcore_axis_name="s"),
    scratch_types=[
        pltpu.VMEM((_NS_W, _CS), jnp.int32),
        pltpu.VMEM((_NS_W * _CS,), jnp.int32),
        pltpu.VMEM((_NS_W, _CS, 32), jnp.float32),
        pltpu.VMEM((_NS_W, _CS), jnp.float32),
        pltpu.SemaphoreType.DMA((_NS_W,)),
        pltpu.SemaphoreType.DMA((_NS_W,)),
        pltpu.SemaphoreType.DMA((_NS_W,)),
    ],
    compiler_params=pltpu.CompilerParams(
        use_tc_tiling_on_sc=False, needs_layout_passes=False
    ),
)
def _gather_kernel(
    tab_hbm, ids_hbm, out_hbm, idx_v, bidx_v, stg_v, row_v, isems, gsems, wsems
):
    wid = lax.axis_index("s") * _NC + lax.axis_index("c")
    r0 = wid * _RPW
    lane = lax.iota(jnp.int32, 16)

    def wave(w, _):
        r = r0 + w // _WPR
        f = r // EMBEDDING_DIM
        d = r % EMBEDDING_DIM
        hoff = (w % _WPR) * (_NS_W * _CS)
        doff = d * _BPR

        ihs = []
        for q in range(_NS_W):
            ihs.append(
                pltpu.async_copy(
                    ids_hbm.at[f, pl.ds(hoff + q * _CS, _CS)], idx_v.at[q], isems.at[q]
                )
            )
        for q in range(_NS_W):
            ihs[q].wait()
            for g in range(_NG):
                sl = pl.ds(g * 16, 16)
                bidx_v[pl.ds(q * _CS + g * 16, 16)] = lax.shift_right_logical(idx_v[q, sl], 5) + doff
        ghs = []
        for q in range(_NS_W):
            ghs.append(
                pltpu.async_copy(
                    tab_hbm.at[bidx_v.at[pl.ds(q * _CS, _CS)]], stg_v.at[q], gsems.at[q]
                )
            )
        whs = []
        for q in range(_NS_W):
            ghs[q].wait()

            def sel(g, _, q=q):
                sl = pl.ds(g * 16, 16)
                low = lax.bitwise_and(idx_v[q, sl], 31)
                row_v[q, sl] = plsc.load_gather(stg_v.at[q], [g * 16 + lane, low])
                return ()

            lax.fori_loop(0, _NG, sel, ())
            whs.append(
                pltpu.async_copy(
                    row_v.at[q], out_hbm.at[f, d, pl.ds(hoff + q * _CS, _CS)], wsems.at[q]
                )
            )
        for q in range(_NS_W):
            whs[q].wait()
        return ()

    lax.fori_loop(0, _NWAVE, wave, ())


@jax.jit
def kernel(relation_ids, embedding_table):
    tab4 = embedding_table.T.reshape(EMBEDDING_DIM * NUM_RELATIONS // 32, 32)
    ids_t = relation_ids.T.astype(jnp.int32)
    out = _gather_kernel(tab4, ids_t)
    return out.transpose(2, 0, 1)


# speedup vs baseline: 3.6276x; 3.6276x over previous
import functools

import jax
import jax.numpy as jnp
from jax import lax
from jax.experimental import pallas as pl
from jax.experimental.pallas import tpu as pltpu
from jax.experimental.pallas import tpu_sc as plsc

NUM_RELATIONS = 1000000
EMBEDDING_DIM = 32
BATCH = 4096
FIELDS = 26

_info = plsc.get_sparse_core_info()
_NC, _NS = _info.num_cores, _info.num_subcores
_NW = _NC * _NS
_B = BATCH * FIELDS
_BPW = _B // _NW
_CS = 128
_NCH = _BPW // _CS
_NBUF = 2
_NG = _CS // 16


@functools.partial(
    pl.kernel,
    out_type=jax.ShapeDtypeStruct((_B, EMBEDDING_DIM), jnp.float32),
    mesh=plsc.VectorSubcoreMesh(core_axis_name="c", subcore_axis_name="s"),
    scratch_types=[
        pltpu.VMEM((_NBUF, _CS), jnp.int32),
        pltpu.VMEM((_NBUF * _CS,), jnp.int32),
        pltpu.VMEM((_NBUF, _CS, 128), jnp.float32),
        pltpu.VMEM((_NBUF, _CS, EMBEDDING_DIM), jnp.float32),
        pltpu.SemaphoreType.DMA((_NBUF,)),
        pltpu.SemaphoreType.DMA((_NBUF,)),
        pltpu.SemaphoreType.DMA((_NBUF,)),
    ],
    compiler_params=pltpu.CompilerParams(
        use_tc_tiling_on_sc=True, needs_layout_passes=False
    ),
)
def _gather_kernel(
    tab_hbm, ids_hbm, out_hbm, idx_v, bidx_v, stg_v, row_v, isems, gsems, wsems
):
    wid = lax.axis_index("s") * _NC + lax.axis_index("c")
    base = wid * _BPW
    lane = lax.iota(jnp.int32, 16)

    ih = [None] * _NCH
    gh = [None] * _NCH
    wh = [None] * _NCH

    def fire(k):
        b = k % _NBUF
        ih[k] = pltpu.async_copy(
            ids_hbm.at[pl.ds(base + k * _CS, _CS)], idx_v.at[b], isems.at[b]
        )
        ih[k].wait()
        for g in range(_NG):
            sl = pl.ds(g * 16, 16)
            bidx_v[pl.ds(b * _CS + g * 16, 16)] = lax.shift_right_logical(
                idx_v[b, sl], 2
            )
        gh[k] = pltpu.async_copy(
            tab_hbm.at[bidx_v.at[pl.ds(b * _CS, _CS)]], stg_v.at[b], gsems.at[b]
        )

    def drain(k):
        b = k % _NBUF
        gh[k].wait()

        def sel(c, _):
            for g in range(_NG):
                sl = pl.ds(g * 16, 16)
                low = lax.bitwise_and(idx_v[b, sl], 3)
                vals = plsc.load_gather(
                    stg_v.at[b], [g * 16 + lane, low * EMBEDDING_DIM + c]
                )
                plsc.store_scatter(
                    row_v.at[b], [g * 16 + lane, jnp.full((16,), c, jnp.int32)], vals
                )
            return ()

        lax.fori_loop(0, EMBEDDING_DIM, sel, ())
        wh[k] = pltpu.async_copy(
            row_v.at[b],
            out_hbm.at[pl.ds(base + k * _CS, _CS)],
            wsems.at[b],
        )

    fire(0)
    for k in range(_NCH):
        if k + 1 < _NCH:
            if k + 1 >= _NBUF:
                wh[k + 1 - _NBUF].wait()
            fire(k + 1)
        drain(k)
    for k in range(_NCH - _NBUF, _NCH):
        wh[k].wait()


@jax.jit
def kernel(relation_ids, embedding_table):
    tab_blk = embedding_table.reshape(NUM_RELATIONS // 4, 4 * EMBEDDING_DIM)
    flat_ids = relation_ids.reshape(-1).astype(jnp.int32)
    out = _gather_kernel(tab_blk, flat_ids)
    return out.reshape(BATCH, FIELDS, EMBEDDING_DIM)


# speedup vs baseline: 4.9391x; 1.3615x over previous
import functools

import jax
import jax.numpy as jnp
from jax import lax
from jax.experimental import pallas as pl
from jax.experimental.pallas import tpu as pltpu
from jax.experimental.pallas import tpu_sc as plsc

NUM_RELATIONS = 1000000
EMBEDDING_DIM = 32
BATCH = 4096
FIELDS = 26

_info = plsc.get_sparse_core_info()
_NC, _NS = _info.num_cores, _info.num_subcores
_NW = _NC * _NS
_B = BATCH * FIELDS
_BPW = _B // _NW
_NCH = 8
_CPW = _BPW // _NCH


@functools.partial(
    pl.kernel,
    out_type=jax.ShapeDtypeStruct((_B, EMBEDDING_DIM), jnp.float32),
    mesh=plsc.VectorSubcoreMesh(core_axis_name="c", subcore_axis_name="s"),
    scratch_types=[
        pltpu.VMEM((_BPW,), jnp.int32),
        pltpu.VMEM((_NCH, _CPW, EMBEDDING_DIM), jnp.float32),
        pltpu.SemaphoreType.DMA((_NCH,)),
        pltpu.SemaphoreType.DMA((_NCH,)),
    ],
    compiler_params=pltpu.CompilerParams(use_tc_tiling_on_sc=False),
)
def _gather_kernel(table_hbm, idx_hbm, out_hbm, idx_v, rows_v, gsems, wsems):
    wid = lax.axis_index("s") * _NC + lax.axis_index("c")
    base = wid * _BPW
    pltpu.sync_copy(idx_hbm.at[pl.ds(base, _BPW)], idx_v)
    gh = []
    for i in range(_NCH):
        gh.append(
            pltpu.async_copy(
                table_hbm.at[idx_v.at[pl.ds(i * _CPW, _CPW)]],
                rows_v.at[i],
                gsems.at[i],
            )
        )
    wh = []
    for i in range(_NCH):
        gh[i].wait()
        wh.append(
            pltpu.async_copy(
                rows_v.at[i],
                out_hbm.at[pl.ds(base + i * _CPW, _CPW)],
                wsems.at[i],
            )
        )
    for i in range(_NCH):
        wh[i].wait()


@jax.jit
def kernel(relation_ids, embedding_table):
    flat_ids = relation_ids.reshape(-1).astype(jnp.int32)
    out = _gather_kernel(embedding_table, flat_ids)
    return out.reshape(BATCH, FIELDS, EMBEDDING_DIM)
